# Initial kernel scaffold; baseline (speedup 1.0000x reference)
import functools

import jax
import jax.numpy as jnp
from jax import lax
from jax.experimental import pallas as pl
from jax.experimental.pallas import tpu as pltpu
from jax.experimental.pallas import tpu_sc as plsc

NC = 2
NS = 16
NW = NC * NS
LANES = 16


def _proj_body(w_ref, x_ref, o_ref):
    o_ref[...] = lax.dot_general(
        x_ref[...], w_ref[...],
        dimension_numbers=(((1,), (1,)), ((), ())),
        preferred_element_type=jnp.float32,
    )


def _project_table(word_table, W, block_rows=8192):
    v, ws = word_table.shape
    d = W.shape[0]
    grid = (v + block_rows - 1) // block_rows
    return pl.pallas_call(
        _proj_body,
        grid=(grid,),
        in_specs=[
            pl.BlockSpec((d, ws), lambda i: (0, 0)),
            pl.BlockSpec((block_rows, ws), lambda i: (i, 0)),
        ],
        out_specs=pl.BlockSpec((block_rows, d), lambda i: (i, 0)),
        out_shape=jax.ShapeDtypeStruct((grid * block_rows, d), jnp.float32),
    )(W, word_table)


def _make_sc_gather_add(n_rows, d, chunk=128):
    per_w = n_rows // NW
    n_chunks = per_w // chunk
    d_vecs = d // LANES
    mesh = plsc.VectorSubcoreMesh(core_axis_name="c", subcore_axis_name="s")

    @functools.partial(
        pl.kernel,
        mesh=mesh,
        out_type=jax.ShapeDtypeStruct((n_rows, d), jnp.float32),
        scratch_types=[
            pltpu.VMEM((per_w,), jnp.int32),
            pltpu.VMEM((per_w,), jnp.int32),
            pltpu.VMEM((chunk, d), jnp.float32),
            pltpu.VMEM((chunk, d), jnp.float32),
            pltpu.SemaphoreType.DMA,
            pltpu.SemaphoreType.DMA,
        ],
    )
    def sc_kernel(proj_hbm, ctab_hbm, widx_hbm, cidx_hbm, out_hbm,
                  widx_v, cidx_v, wrows, crows, sem_w, sem_c):
        wid = lax.axis_index("s") * NC + lax.axis_index("c")
        base = wid * per_w
        pltpu.sync_copy(widx_hbm.at[pl.ds(base, per_w)], widx_v)
        pltpu.sync_copy(cidx_hbm.at[pl.ds(base, per_w)], cidx_v)

        def chunk_body(c, carry):
            off = c * chunk
            cp_w = pltpu.async_copy(
                proj_hbm.at[widx_v.at[pl.ds(off, chunk)]], wrows, sem_w)
            cp_c = pltpu.async_copy(
                ctab_hbm.at[cidx_v.at[pl.ds(off, chunk)]], crows, sem_c)
            cp_w.wait()
            cp_c.wait()

            def add_row(j, carry2):
                for k in range(d_vecs):
                    sl = pl.ds(k * LANES, LANES)
                    wrows[j, sl] = wrows[j, sl] + crows[j, sl]
                return carry2

            lax.fori_loop(0, chunk, add_row, 0, unroll=4)
            pltpu.sync_copy(wrows, out_hbm.at[pl.ds(base + off, chunk)])
            return carry

        lax.fori_loop(0, n_chunks, chunk_body, 0)

    return sc_kernel


def kernel(char_ids, word_ids, char_table, word_table, W):
    b, l = char_ids.shape
    d = char_table.shape[1]
    n = b * l

    proj = _project_table(word_table, W)
    sc = _make_sc_gather_add(n, d)
    out_flat = sc(proj, char_table,
                  word_ids.reshape(-1).astype(jnp.int32),
                  char_ids.reshape(-1).astype(jnp.int32))
    return out_flat.reshape(b, l, d)


# speedup vs baseline: 1.8863x; 1.8863x over previous
import functools

import jax
import jax.numpy as jnp
from jax import lax
from jax.experimental import pallas as pl
from jax.experimental.pallas import tpu as pltpu
from jax.experimental.pallas import tpu_sc as plsc

NC = 2
NS = 16
NW = NC * NS
LANES = 16


def _proj_body(w_ref, x_ref, o_ref):
    o_ref[...] = lax.dot_general(
        x_ref[...], w_ref[...],
        dimension_numbers=(((1,), (1,)), ((), ())),
        preferred_element_type=jnp.float32,
    )


def _project_table(word_table, W, block_rows=8192):
    v, ws = word_table.shape
    d = W.shape[0]
    grid = (v + block_rows - 1) // block_rows
    return pl.pallas_call(
        _proj_body,
        grid=(grid,),
        in_specs=[
            pl.BlockSpec((d, ws), lambda i: (0, 0)),
            pl.BlockSpec((block_rows, ws), lambda i: (i, 0)),
        ],
        out_specs=pl.BlockSpec((block_rows, d), lambda i: (i, 0)),
        out_shape=jax.ShapeDtypeStruct((grid * block_rows, d), jnp.float32),
    )(W, word_table)


def _make_sc_gather_add(n_rows, d, chunk=128):
    per_w = n_rows // NW
    n_chunks = per_w // chunk
    d_vecs = d // LANES
    mesh = plsc.VectorSubcoreMesh(core_axis_name="c", subcore_axis_name="s")

    @functools.partial(
        pl.kernel,
        mesh=mesh,
        out_type=jax.ShapeDtypeStruct((n_rows, d), jnp.float32),
        scratch_types=[
            pltpu.VMEM((per_w,), jnp.int32),
            pltpu.VMEM((per_w,), jnp.int32),
            pltpu.VMEM((chunk, d), jnp.float32),
            pltpu.VMEM((chunk, d), jnp.float32),
            pltpu.SemaphoreType.DMA,
            pltpu.SemaphoreType.DMA,
        ],
        compiler_params=pltpu.CompilerParams(use_tc_tiling_on_sc=False),
    )
    def sc_kernel(proj_hbm, ctab_hbm, widx_hbm, cidx_hbm, out_hbm,
                  widx_v, cidx_v, wrows, crows, sem_w, sem_c):
        wid = lax.axis_index("s") * NC + lax.axis_index("c")
        base = wid * per_w
        pltpu.sync_copy(widx_hbm.at[pl.ds(base, per_w)], widx_v)
        pltpu.sync_copy(cidx_hbm.at[pl.ds(base, per_w)], cidx_v)

        def chunk_body(c, carry):
            off = c * chunk
            cp_w = pltpu.async_copy(
                proj_hbm.at[widx_v.at[pl.ds(off, chunk)]], wrows, sem_w)
            cp_c = pltpu.async_copy(
                ctab_hbm.at[cidx_v.at[pl.ds(off, chunk)]], crows, sem_c)
            cp_w.wait()
            cp_c.wait()

            def add_row(j, carry2):
                for k in range(d_vecs):
                    sl = pl.ds(k * LANES, LANES)
                    wrows[j, sl] = wrows[j, sl] + crows[j, sl]
                return carry2

            lax.fori_loop(0, chunk, add_row, 0, unroll=4)
            pltpu.sync_copy(wrows, out_hbm.at[pl.ds(base + off, chunk)])
            return carry

        lax.fori_loop(0, n_chunks, chunk_body, 0)

    return sc_kernel


def kernel(char_ids, word_ids, char_table, word_table, W):
    b, l = char_ids.shape
    d = char_table.shape[1]
    n = b * l

    proj = _project_table(word_table, W)
    sc = _make_sc_gather_add(n, d)
    out_flat = sc(proj, char_table,
                  word_ids.reshape(-1).astype(jnp.int32),
                  char_ids.reshape(-1).astype(jnp.int32))
    return out_flat.reshape(b, l, d)


# speedup vs baseline: 2.0975x; 1.1120x over previous
import functools

import jax
import jax.numpy as jnp
from jax import lax
from jax.experimental import pallas as pl
from jax.experimental.pallas import tpu as pltpu
from jax.experimental.pallas import tpu_sc as plsc

NC = 2
NS = 16
NW = NC * NS
LANES = 16


def _proj_body(w_ref, x_ref, o_ref):
    o_ref[...] = lax.dot_general(
        x_ref[...], w_ref[...],
        dimension_numbers=(((1,), (1,)), ((), ())),
        preferred_element_type=jnp.float32,
    )


def _project_table(word_table, W, block_rows=8192):
    v, ws = word_table.shape
    d = W.shape[0]
    grid = (v + block_rows - 1) // block_rows
    return pl.pallas_call(
        _proj_body,
        grid=(grid,),
        in_specs=[
            pl.BlockSpec((d, ws), lambda i: (0, 0)),
            pl.BlockSpec((block_rows, ws), lambda i: (i, 0)),
        ],
        out_specs=pl.BlockSpec((block_rows, d), lambda i: (i, 0)),
        out_shape=jax.ShapeDtypeStruct((grid * block_rows, d), jnp.float32),
    )(W, word_table)


def _make_sc_gather_add(n_rows, d, chunk=128):
    per_w = n_rows // NW
    n_chunks = per_w // chunk
    n_rounds = n_chunks // 2
    d_vecs = d // LANES
    mesh = plsc.VectorSubcoreMesh(core_axis_name="c", subcore_axis_name="s")

    @functools.partial(
        pl.kernel,
        mesh=mesh,
        out_type=jax.ShapeDtypeStruct((n_rows, d), jnp.float32),
        scratch_types=[
            pltpu.VMEM((per_w,), jnp.int32),
            pltpu.VMEM((per_w,), jnp.int32),
            [pltpu.VMEM((chunk, d), jnp.float32) for _ in range(2)],
            [pltpu.VMEM((chunk, d), jnp.float32) for _ in range(2)],
            [pltpu.VMEM((chunk, d), jnp.float32) for _ in range(2)],
            [pltpu.SemaphoreType.DMA for _ in range(2)],
            [pltpu.SemaphoreType.DMA for _ in range(2)],
            [pltpu.SemaphoreType.DMA for _ in range(2)],
        ],
        compiler_params=pltpu.CompilerParams(use_tc_tiling_on_sc=False),
    )
    def sc_kernel(proj_hbm, ctab_hbm, widx_hbm, cidx_hbm, out_hbm,
                  widx_v, cidx_v, wrows, crows, obuf, gw, gc, ss):
        wid = lax.axis_index("s") * NC + lax.axis_index("c")
        base = wid * per_w
        pltpu.sync_copy(widx_hbm.at[pl.ds(base, per_w)], widx_v)
        pltpu.sync_copy(cidx_hbm.at[pl.ds(base, per_w)], cidx_v)

        def gathers(c, b):
            pltpu.async_copy(
                proj_hbm.at[widx_v.at[pl.ds(c * chunk, chunk)]], wrows[b], gw[b])
            pltpu.async_copy(
                ctab_hbm.at[cidx_v.at[pl.ds(c * chunk, chunk)]], crows[b], gc[b])

        gathers(0, 0)
        gathers(1, 1)

        def round_body(r, carry):
            for b in range(2):
                c = r * 2 + b
                off = c * chunk
                pltpu.make_async_copy(
                    proj_hbm.at[widx_v.at[pl.ds(off, chunk)]], wrows[b], gw[b]
                ).wait()
                pltpu.make_async_copy(
                    ctab_hbm.at[cidx_v.at[pl.ds(off, chunk)]], crows[b], gc[b]
                ).wait()

                @pl.when(r >= 1)
                def _():
                    pltpu.make_async_copy(
                        obuf[b],
                        out_hbm.at[pl.ds(base + off - 2 * chunk, chunk)],
                        ss[b],
                    ).wait()

                def add_row(j, carry2):
                    for k in range(d_vecs):
                        sl = pl.ds(k * LANES, LANES)
                        obuf[b][j, sl] = wrows[b][j, sl] + crows[b][j, sl]
                    return carry2

                lax.fori_loop(0, chunk, add_row, 0, unroll=4)

                @pl.when(r < n_rounds - 1)
                def _():
                    gathers(c + 2, b)

                pltpu.async_copy(
                    obuf[b], out_hbm.at[pl.ds(base + off, chunk)], ss[b])
            return carry

        lax.fori_loop(0, n_rounds, round_body, 0)

        for b in range(2):
            off = (n_chunks - 2 + b) * chunk
            pltpu.make_async_copy(
                obuf[b], out_hbm.at[pl.ds(base + off, chunk)], ss[b]).wait()

    return sc_kernel


def kernel(char_ids, word_ids, char_table, word_table, W):
    b, l = char_ids.shape
    d = char_table.shape[1]
    n = b * l

    proj = _project_table(word_table, W)
    sc = _make_sc_gather_add(n, d)
    out_flat = sc(proj, char_table,
                  word_ids.reshape(-1).astype(jnp.int32),
                  char_ids.reshape(-1).astype(jnp.int32))
    return out_flat.reshape(b, l, d)
